# baseline (speedup 1.0000x reference)
import functools

import jax
import jax.numpy as jnp
from jax import lax
from jax.experimental import pallas as pl
from jax.experimental.pallas import tpu as pltpu
from jax.experimental.pallas import tpu_sc as plsc

NBUF = 2


def kernel(focuses, mask, pe):
    B, L = focuses.shape
    V, H = pe.shape
    info = plsc.get_sparse_core_info()
    nc, ns = info.num_cores, info.num_subcores
    nw = nc * ns
    HP = H // 2
    WB = B // 2
    NHO = H // 8
    NT = WB // 128
    n_units = L // 2
    assert nw == 2 * 2 * NHO and L % 2 == 0 and B % 256 == 0

    pe_u16 = jax.lax.bitcast_convert_type(
        pe.astype(jnp.bfloat16), jnp.uint16
    ).astype(jnp.uint32)
    lo = pe_u16[:, 0::2]
    hi = pe_u16[:, 1::2]
    pe_pack = jax.lax.bitcast_convert_type(lo | (hi << 16), jnp.int32)
    pe_flat = pe_pack.reshape(V * HP)

    focT = focuses.T
    mskT = mask.T.astype(jnp.int32)

    @functools.partial(
        pl.kernel,
        mesh=plsc.VectorSubcoreMesh(core_axis_name="c", subcore_axis_name="s"),
        compiler_params=pltpu.CompilerParams(use_tc_tiling_on_sc=False, needs_layout_passes=False),
        out_type=jax.ShapeDtypeStruct((L, NHO, B // 128, 8, 128), jnp.float32),
        scratch_types=[
            pltpu.VMEM((V * HP,), jnp.int32),
            pltpu.VMEM((NBUF, WB), jnp.int32),
            pltpu.VMEM((NBUF, WB), jnp.int32),
            pltpu.VMEM((NBUF, NT, 8, 128), jnp.float32),
        ]
        + [pltpu.SemaphoreType.DMA] * (3 * NBUF),
    )
    def fe_kernel(pe_hbm, foc_hbm, msk_hbm, out_hbm, tab_v, idx_v, msk_v,
                  slab_v, *sems):
        isems = sems[:NBUF]
        wsems = sems[NBUF : 2 * NBUF]
        msems = sems[2 * NBUF :]
        c = lax.axis_index("c")
        s = lax.axis_index("s")
        wid = s * nc + c
        ho = wid % NHO
        half = (wid // NHO) % 2
        lpar = wid // (2 * NHO)
        bco = half * WB

        pltpu.sync_copy(pe_hbm, tab_v)

        def stage(u, slot):
            l = lpar + 2 * u
            pltpu.async_copy(
                foc_hbm.at[l, pl.ds(bco, WB)], idx_v.at[slot], isems[slot]
            )
            pltpu.async_copy(
                msk_hbm.at[l, pl.ds(bco, WB)], msk_v.at[slot], msems[slot]
            )

        def wait_stage(u, slot):
            l = lpar + 2 * u
            pltpu.make_async_copy(
                foc_hbm.at[l, pl.ds(bco, WB)], idx_v.at[slot], isems[slot]
            ).wait()
            pltpu.make_async_copy(
                msk_hbm.at[l, pl.ds(bco, WB)], msk_v.at[slot], msems[slot]
            ).wait()

        def out_slice(u):
            l = lpar + 2 * u
            return out_hbm.at[l, ho, pl.ds(NT * half, NT)]

        for b in range(NBUF):
            stage(b, b)

        def unit(u, carry):
            for b in range(NBUF):
                uu = u * NBUF + b

                @pl.when(uu >= NBUF)
                def _():
                    pltpu.make_async_copy(
                        slab_v.at[b], out_slice(uu - NBUF), wsems[b]
                    ).wait()

                wait_stage(uu, b)

                @plsc.parallel_loop(0, WB // 16, unroll=8)
                def bvec(jv):
                    bt = jv // 8
                    bs = (jv % 8) * 16
                    k = jv * 16
                    f16 = idx_v[b, pl.ds(k, 16)]
                    m16 = msk_v[b, pl.ds(k, 16)]
                    mneg = jnp.int32(0) - m16
                    hmask = mneg & jnp.int32(-65536)
                    addr = (f16 << 5) + (4 * ho)
                    for p in range(4):
                        w = plsc.load_gather(tab_v, [addr + p])
                        lo16 = (w << 16) & mneg
                        hi16 = w & hmask
                        slab_v[b, bt, 2 * p, pl.ds(bs, 16)] = plsc.bitcast(
                            lo16, jnp.float32
                        )
                        slab_v[b, bt, 2 * p + 1, pl.ds(bs, 16)] = plsc.bitcast(
                            hi16, jnp.float32
                        )

                pltpu.async_copy(slab_v.at[b], out_slice(uu), wsems[b])

                @pl.when(uu + NBUF < n_units)
                def _():
                    stage(uu + NBUF, b)

            return carry

        lax.fori_loop(0, n_units // NBUF, unit, 0)

        for b in range(NBUF):
            uu = n_units - NBUF + b
            pltpu.make_async_copy(
                slab_v.at[b], out_slice(uu), wsems[b]
            ).wait()

    out = fe_kernel(pe_flat, focT, mskT)
    return jnp.transpose(out, (2, 4, 0, 1, 3)).reshape(B, L, H)


# speedup vs baseline: 1.9126x; 1.9126x over previous
import functools

import jax
import jax.numpy as jnp
from jax import lax
from jax.experimental import pallas as pl
from jax.experimental.pallas import tpu as pltpu
from jax.experimental.pallas import tpu_sc as plsc

NBUF = 2


def kernel(focuses, mask, pe):
    B, L = focuses.shape
    V, H = pe.shape
    info = plsc.get_sparse_core_info()
    nc, ns = info.num_cores, info.num_subcores
    nw = nc * ns
    HP = H // 2
    WB = B // 2
    NHO = H // 8
    NT = WB // 128
    n_units = L // 2
    assert nw == 2 * 2 * NHO and L % 2 == 0 and B % 256 == 0

    pe_u16 = jax.lax.bitcast_convert_type(
        pe.astype(jnp.bfloat16), jnp.uint16
    ).astype(jnp.uint32)
    lo = pe_u16[:, 0::2]
    hi = pe_u16[:, 1::2]
    pe_pack = jax.lax.bitcast_convert_type(lo | (hi << 16), jnp.int32)
    cols = jnp.arange(HP, dtype=jnp.int32)[None, :] ^ (
        jnp.arange(V, dtype=jnp.int32)[:, None] % HP
    )
    pe_sw = jnp.take_along_axis(pe_pack, cols, axis=1)
    pe_flat = pe_sw.reshape(V * HP)

    focT = focuses.T
    mskT = mask.T.astype(jnp.int32)

    @functools.partial(
        pl.kernel,
        mesh=plsc.VectorSubcoreMesh(core_axis_name="c", subcore_axis_name="s"),
        compiler_params=pltpu.CompilerParams(use_tc_tiling_on_sc=False, needs_layout_passes=False),
        out_type=jax.ShapeDtypeStruct((L, NHO, B // 128, 8, 128), jnp.float32),
        scratch_types=[
            pltpu.VMEM((V * HP,), jnp.int32),
            pltpu.VMEM((NBUF, WB), jnp.int32),
            pltpu.VMEM((NBUF, WB), jnp.int32),
            pltpu.VMEM((NBUF, NT, 8, 128), jnp.float32),
        ]
        + [pltpu.SemaphoreType.DMA] * (3 * NBUF),
    )
    def fe_kernel(pe_hbm, foc_hbm, msk_hbm, out_hbm, tab_v, idx_v, msk_v,
                  slab_v, *sems):
        isems = sems[:NBUF]
        wsems = sems[NBUF : 2 * NBUF]
        msems = sems[2 * NBUF :]
        c = lax.axis_index("c")
        s = lax.axis_index("s")
        wid = s * nc + c
        ho = wid % NHO
        half = (wid // NHO) % 2
        lpar = wid // (2 * NHO)
        bco = half * WB

        pltpu.sync_copy(pe_hbm, tab_v)

        def stage(u, slot):
            l = lpar + 2 * u
            pltpu.async_copy(
                foc_hbm.at[l, pl.ds(bco, WB)], idx_v.at[slot], isems[slot]
            )
            pltpu.async_copy(
                msk_hbm.at[l, pl.ds(bco, WB)], msk_v.at[slot], msems[slot]
            )

        def wait_stage(u, slot):
            l = lpar + 2 * u
            pltpu.make_async_copy(
                foc_hbm.at[l, pl.ds(bco, WB)], idx_v.at[slot], isems[slot]
            ).wait()
            pltpu.make_async_copy(
                msk_hbm.at[l, pl.ds(bco, WB)], msk_v.at[slot], msems[slot]
            ).wait()

        def out_slice(u):
            l = lpar + 2 * u
            return out_hbm.at[l, ho, pl.ds(NT * half, NT)]

        for b in range(NBUF):
            stage(b, b)

        def unit(u, carry):
            for b in range(NBUF):
                uu = u * NBUF + b

                @pl.when(uu >= NBUF)
                def _():
                    pltpu.make_async_copy(
                        slab_v.at[b], out_slice(uu - NBUF), wsems[b]
                    ).wait()

                wait_stage(uu, b)

                @plsc.parallel_loop(0, NT, unroll=2)
                def bvec(bt):
                    for jj in range(8):
                        bs = jj * 16
                        k = bt * 128 + bs
                        f16 = idx_v[b, pl.ds(k, 16)]
                        m16 = msk_v[b, pl.ds(k, 16)]
                        mneg = jnp.int32(0) - m16
                        hmask = mneg & jnp.int32(-65536)
                        base = f16 << 5
                        fx = f16 & jnp.int32(HP - 1)
                        for p in range(4):
                            addr = base + (fx ^ (4 * ho + p))
                            w = plsc.load_gather(tab_v, [addr])
                            lo16 = (w << 16) & mneg
                            hi16 = w & hmask
                            slab_v[b, bt, 2 * p, pl.ds(bs, 16)] = plsc.bitcast(
                                lo16, jnp.float32
                            )
                            slab_v[b, bt, 2 * p + 1, pl.ds(bs, 16)] = plsc.bitcast(
                                hi16, jnp.float32
                            )

                pltpu.async_copy(slab_v.at[b], out_slice(uu), wsems[b])

                @pl.when(uu + NBUF < n_units)
                def _():
                    stage(uu + NBUF, b)

            return carry

        lax.fori_loop(0, n_units // NBUF, unit, 0)

        for b in range(NBUF):
            uu = n_units - NBUF + b
            pltpu.make_async_copy(
                slab_v.at[b], out_slice(uu), wsems[b]
            ).wait()

    out = fe_kernel(pe_flat, focT, mskT)
    return jnp.transpose(out, (2, 4, 0, 1, 3)).reshape(B, L, H)


# speedup vs baseline: 2.8171x; 1.4729x over previous
import functools

import jax
import jax.numpy as jnp
from jax import lax
from jax.experimental import pallas as pl
from jax.experimental.pallas import tpu as pltpu
from jax.experimental.pallas import tpu_sc as plsc

NBUF = 2


def kernel(focuses, mask, pe):
    B, L = focuses.shape
    V, H = pe.shape
    info = plsc.get_sparse_core_info()
    nc, ns = info.num_cores, info.num_subcores
    nw = nc * ns
    HP = H // 2
    WB = B // 2
    NHO = H // 8
    NT = WB // 128
    n_units = L // 2
    assert nw == 2 * 2 * NHO and L % 2 == 0 and B % 256 == 0

    pe_u16 = jax.lax.bitcast_convert_type(
        pe.astype(jnp.bfloat16), jnp.uint16
    ).astype(jnp.uint32)
    lo = pe_u16[:, 0::2]
    hi = pe_u16[:, 1::2]
    pe_pack = jax.lax.bitcast_convert_type(lo | (hi << 16), jnp.int32)
    cols = jnp.arange(HP, dtype=jnp.int32)[None, :] ^ (
        jnp.arange(V, dtype=jnp.int32)[:, None] % HP
    )
    pe_sw = jnp.take_along_axis(pe_pack, cols, axis=1)
    pe_flat = pe_sw.reshape(V * HP)

    focT = focuses.T
    mskT = mask.T.astype(jnp.int32)

    @functools.partial(
        pl.kernel,
        mesh=plsc.VectorSubcoreMesh(core_axis_name="c", subcore_axis_name="s"),
        compiler_params=pltpu.CompilerParams(use_tc_tiling_on_sc=False, needs_layout_passes=False),
        out_type=jax.ShapeDtypeStruct((L, NHO, B // 128, 8, 128), jnp.float32),
        scratch_types=[
            pltpu.VMEM((V * HP,), jnp.int32),
            pltpu.VMEM((NBUF, WB), jnp.int32),
            pltpu.VMEM((NBUF, WB), jnp.int32),
            pltpu.VMEM((NBUF, NT, 8, 128), jnp.float32),
        ]
        + [pltpu.SemaphoreType.DMA] * (3 * NBUF),
    )
    def fe_kernel(pe_hbm, foc_hbm, msk_hbm, out_hbm, tab_v, idx_v, msk_v,
                  slab_v, *sems):
        isems = sems[:NBUF]
        wsems = sems[NBUF : 2 * NBUF]
        msems = sems[2 * NBUF :]
        c = lax.axis_index("c")
        s = lax.axis_index("s")
        wid = s * nc + c
        ho = wid % NHO
        half = (wid // NHO) % 2
        lpar = wid // (2 * NHO)
        bco = half * WB

        pltpu.sync_copy(pe_hbm, tab_v)

        def stage(u, slot):
            l = lpar + 2 * u
            pltpu.async_copy(
                foc_hbm.at[l, pl.ds(bco, WB)], idx_v.at[slot], isems[slot]
            )
            pltpu.async_copy(
                msk_hbm.at[l, pl.ds(bco, WB)], msk_v.at[slot], msems[slot]
            )

        def wait_stage(u, slot):
            l = lpar + 2 * u
            pltpu.make_async_copy(
                foc_hbm.at[l, pl.ds(bco, WB)], idx_v.at[slot], isems[slot]
            ).wait()
            pltpu.make_async_copy(
                msk_hbm.at[l, pl.ds(bco, WB)], msk_v.at[slot], msems[slot]
            ).wait()

        def out_slice(u):
            l = lpar + 2 * u
            return out_hbm.at[l, ho, pl.ds(NT * half, NT)]

        for b in range(NBUF):
            stage(b, b)

        def unit(u, carry):
            for b in range(NBUF):
                uu = u * NBUF + b

                @pl.when(uu >= NBUF)
                def _():
                    pltpu.make_async_copy(
                        slab_v.at[b], out_slice(uu - NBUF), wsems[b]
                    ).wait()

                wait_stage(uu, b)

                @plsc.parallel_loop(0, NT, unroll=1)
                def bvec(bt):
                    for jj in range(8):
                        bs = jj * 16
                        k = bt * 128 + bs
                        f16 = idx_v[b, pl.ds(k, 16)]
                        m16 = msk_v[b, pl.ds(k, 16)]
                        mneg = jnp.int32(0) - m16
                        hmask = mneg & jnp.int32(-65536)
                        base = f16 << 5
                        fx = f16 & jnp.int32(HP - 1)
                        for p in range(4):
                            addr = base + (fx ^ (4 * ho + p))
                            w = plsc.load_gather(tab_v, [addr])
                            lo16 = (w << 16) & mneg
                            hi16 = w & hmask
                            slab_v[b, bt, 2 * p, pl.ds(bs, 16)] = plsc.bitcast(
                                lo16, jnp.float32
                            )
                            slab_v[b, bt, 2 * p + 1, pl.ds(bs, 16)] = plsc.bitcast(
                                hi16, jnp.float32
                            )

                pltpu.async_copy(slab_v.at[b], out_slice(uu), wsems[b])

                @pl.when(uu + NBUF < n_units)
                def _():
                    stage(uu + NBUF, b)

            return carry

        lax.fori_loop(0, n_units // NBUF, unit, 0)

        for b in range(NBUF):
            uu = n_units - NBUF + b
            pltpu.make_async_copy(
                slab_v.at[b], out_slice(uu), wsems[b]
            ).wait()

    out = fe_kernel(pe_flat, focT, mskT)
    return jnp.transpose(out, (2, 4, 0, 1, 3)).reshape(B, L, H)
